# initial kernel scaffold (unmeasured)
import jax
import jax.numpy as jnp
from jax import lax
from jax.experimental import pallas as pl
from jax.experimental.pallas import tpu as pltpu

N_DEV = 4
M_BLK = 1024
K_BLK = 1024
K_ALL = 4096
N_OUT = 2048
N_TILE = 512


def kernel(x, w_mat):
    def body(x_hbm, w_ref, out_ref, recv_buf, amax_src, amax_recv,
             local_sem, send_sems, recv_sems, amax_send_sems, amax_recv_sems):
        my_i = lax.axis_index("i")

        barrier = pltpu.get_barrier_semaphore()
        for o in (1, 2, 3):
            peer = lax.rem(my_i + o, N_DEV)
            pl.semaphore_signal(barrier, inc=1, device_id=(peer,),
                                device_id_type=pl.DeviceIdType.MESH)
        pl.semaphore_wait(barrier, N_DEV - 1)

        local_cp = pltpu.make_async_copy(
            x_hbm.at[pl.ds(my_i * M_BLK, M_BLK), :], recv_buf.at[0], local_sem)
        local_cp.start()

        rdmas = {}
        for o in (2, 1, 3):
            t = lax.rem(my_i + o, N_DEV)
            rdma = pltpu.make_async_remote_copy(
                src_ref=x_hbm.at[pl.ds(t * M_BLK, M_BLK), :],
                dst_ref=recv_buf.at[o],
                send_sem=send_sems.at[o - 1],
                recv_sem=recv_sems.at[o - 1],
                device_id=(t,),
                device_id_type=pl.DeviceIdType.MESH)
            rdma.start()
            rdmas[o] = rdma

        def accum_block(slot, src_idx, first):
            for nt in range(0, N_OUT, N_TILE):
                part = jnp.dot(
                    recv_buf[slot],
                    w_ref[pl.ds(src_idx * K_BLK, K_BLK), nt:nt + N_TILE],
                    precision=lax.Precision.HIGH,
                    preferred_element_type=jnp.float32)
                if first:
                    out_ref[:, nt:nt + N_TILE] = part
                else:
                    out_ref[:, nt:nt + N_TILE] += part

        local_cp.wait()
        accum_block(0, my_i, first=True)

        for o in (1, 3, 2):
            rdmas[o].wait_recv()
            src = lax.rem(my_i - o + N_DEV, N_DEV)
            accum_block(o, src, first=False)

        for o in (1, 2, 3):
            rdmas[o].wait_send()

        local_amax = jnp.maximum(jnp.max(out_ref[...]), 0.0)
        amax_src[...] = jnp.full((8, 128), local_amax, jnp.float32)
        a_rdmas = []
        for o in (1, 2, 3):
            t = lax.rem(my_i + o, N_DEV)
            r = pltpu.make_async_remote_copy(
                src_ref=amax_src,
                dst_ref=amax_recv.at[o - 1],
                send_sem=amax_send_sems.at[o - 1],
                recv_sem=amax_recv_sems.at[o - 1],
                device_id=(t,),
                device_id_type=pl.DeviceIdType.MESH)
            r.start()
            a_rdmas.append(r)
        for r in a_rdmas:
            r.wait()
        g_amax = jnp.maximum(local_amax, jnp.max(amax_recv[...]))

        scale = g_amax / 127.0
        inv = 127.0 / g_amax
        for nt in range(0, N_OUT, N_TILE):
            y = jnp.maximum(out_ref[:, nt:nt + N_TILE], 0.0)
            q = jnp.clip(jnp.round(y * inv), -127.0, 127.0)
            out_ref[:, nt:nt + N_TILE] = q * scale

    return pl.pallas_call(
        body,
        out_shape=jax.ShapeDtypeStruct((M_BLK, N_OUT), jnp.float32),
        in_specs=[
            pl.BlockSpec(memory_space=pltpu.ANY),
            pl.BlockSpec(memory_space=pltpu.VMEM),
        ],
        out_specs=pl.BlockSpec(memory_space=pltpu.VMEM),
        scratch_shapes=[
            pltpu.VMEM((N_DEV, M_BLK, K_BLK), jnp.float32),
            pltpu.VMEM((8, 128), jnp.float32),
            pltpu.VMEM((3, 8, 128), jnp.float32),
            pltpu.SemaphoreType.DMA,
            pltpu.SemaphoreType.DMA((3,)),
            pltpu.SemaphoreType.DMA((3,)),
            pltpu.SemaphoreType.DMA((3,)),
            pltpu.SemaphoreType.DMA((3,)),
        ],
        compiler_params=pltpu.CompilerParams(collective_id=0),
    )(x, w_mat)


# baseline (device time: 206303 ns/iter reference)
import jax
import jax.numpy as jnp
from jax import lax
from jax.experimental import pallas as pl
from jax.experimental.pallas import tpu as pltpu

N_DEV = 4


def kernel(x, w_mat):
    M_BLK = x.shape[0] // N_DEV
    K_BLK = x.shape[1]
    N_OUT = w_mat.shape[1]
    N_TILE = min(512, N_OUT)
    def body(x_hbm, w_ref, out_ref, recv_buf, amax_src, amax_recv,
             local_sem, send_sems, recv_sems, amax_send_sems, amax_recv_sems):
        my_i = lax.axis_index("i")

        barrier = pltpu.get_barrier_semaphore()
        for o in (1, 2, 3):
            peer = lax.rem(my_i + o, N_DEV)
            pl.semaphore_signal(barrier, inc=1, device_id=(peer,),
                                device_id_type=pl.DeviceIdType.MESH)
        pl.semaphore_wait(barrier, N_DEV - 1)

        local_cp = pltpu.make_async_copy(
            x_hbm.at[pl.ds(my_i * M_BLK, M_BLK), :], recv_buf.at[0], local_sem)
        local_cp.start()

        rdmas = {}
        for o in (2, 1, 3):
            t = lax.rem(my_i + o, N_DEV)
            rdma = pltpu.make_async_remote_copy(
                src_ref=x_hbm.at[pl.ds(t * M_BLK, M_BLK), :],
                dst_ref=recv_buf.at[o],
                send_sem=send_sems.at[o - 1],
                recv_sem=recv_sems.at[o - 1],
                device_id=(t,),
                device_id_type=pl.DeviceIdType.MESH)
            rdma.start()
            rdmas[o] = rdma

        def accum_block(slot, src_idx, first):
            for nt in range(0, N_OUT, N_TILE):
                part = jnp.dot(
                    recv_buf[slot],
                    w_ref[pl.ds(src_idx * K_BLK, K_BLK), nt:nt + N_TILE],
                    precision=lax.Precision.HIGHEST,
                    preferred_element_type=jnp.float32)
                if first:
                    out_ref[:, nt:nt + N_TILE] = part
                else:
                    out_ref[:, nt:nt + N_TILE] += part

        local_cp.wait()
        accum_block(0, my_i, first=True)

        for o in (1, 3, 2):
            rdmas[o].wait_recv()
            src = lax.rem(my_i - o + N_DEV, N_DEV)
            accum_block(o, src, first=False)

        for o in (1, 2, 3):
            rdmas[o].wait_send()

        local_amax = jnp.maximum(jnp.max(out_ref[...]), 0.0)
        amax_src[...] = jnp.full((8, 128), local_amax, jnp.float32)
        a_rdmas = []
        for o in (1, 2, 3):
            t = lax.rem(my_i + o, N_DEV)
            r = pltpu.make_async_remote_copy(
                src_ref=amax_src,
                dst_ref=amax_recv.at[o - 1],
                send_sem=amax_send_sems.at[o - 1],
                recv_sem=amax_recv_sems.at[o - 1],
                device_id=(t,),
                device_id_type=pl.DeviceIdType.MESH)
            r.start()
            a_rdmas.append(r)
        for r in a_rdmas:
            r.wait()
        g_amax = jnp.maximum(local_amax, jnp.max(amax_recv[...]))

        scale = g_amax / 127.0
        inv = 127.0 / g_amax
        for nt in range(0, N_OUT, N_TILE):
            y = jnp.maximum(out_ref[:, nt:nt + N_TILE], 0.0)
            q = jnp.clip(jnp.round(y * inv), -127.0, 127.0)
            out_ref[:, nt:nt + N_TILE] = q * scale

    return pl.pallas_call(
        body,
        out_shape=jax.ShapeDtypeStruct((M_BLK, N_OUT), jnp.float32),
        in_specs=[
            pl.BlockSpec(memory_space=pl.ANY),
            pl.BlockSpec(memory_space=pltpu.VMEM),
        ],
        out_specs=pl.BlockSpec(memory_space=pltpu.VMEM),
        scratch_shapes=[
            pltpu.VMEM((N_DEV, M_BLK, K_BLK), jnp.float32),
            pltpu.VMEM((8, 128), jnp.float32),
            pltpu.VMEM((3, 8, 128), jnp.float32),
            pltpu.SemaphoreType.DMA,
            pltpu.SemaphoreType.DMA((3,)),
            pltpu.SemaphoreType.DMA((3,)),
            pltpu.SemaphoreType.DMA((3,)),
            pltpu.SemaphoreType.DMA((3,)),
        ],
        compiler_params=pltpu.CompilerParams(
            collective_id=0, vmem_limit_bytes=100 * 1024 * 1024),
    )(x, w_mat)


# device time: 115140 ns/iter; 1.7918x vs baseline; 1.7918x over previous
import jax
import jax.numpy as jnp
from jax import lax
from jax.experimental import pallas as pl
from jax.experimental.pallas import tpu as pltpu

N_DEV = 4


def kernel(x, w_mat):
    M_BLK = x.shape[0] // N_DEV
    K_BLK = x.shape[1]
    N_OUT = w_mat.shape[1]

    x_bf = x.astype(jnp.bfloat16)
    w_bf = w_mat.astype(jnp.bfloat16)

    def body(x_hbm, w_ref, out_ref, recv_buf, amax_src, amax_recv,
             local_sem, send_sems, recv_sems, amax_send_sems, amax_recv_sems):
        my_i = lax.axis_index("i")

        barrier = pltpu.get_barrier_semaphore()
        for o in (1, 2, 3):
            peer = lax.rem(my_i + o, N_DEV)
            pl.semaphore_signal(barrier, inc=1, device_id=(peer,),
                                device_id_type=pl.DeviceIdType.MESH)
        pl.semaphore_wait(barrier, N_DEV - 1)

        local_cp = pltpu.make_async_copy(
            x_hbm.at[pl.ds(my_i * M_BLK, M_BLK), :], recv_buf.at[0], local_sem)
        local_cp.start()

        rdmas = {}
        for o in (2, 1, 3):
            t = lax.rem(my_i + o, N_DEV)
            rdma = pltpu.make_async_remote_copy(
                src_ref=x_hbm.at[pl.ds(t * M_BLK, M_BLK), :],
                dst_ref=recv_buf.at[o],
                send_sem=send_sems.at[o - 1],
                recv_sem=recv_sems.at[o - 1],
                device_id=(t,),
                device_id_type=pl.DeviceIdType.MESH)
            rdma.start()
            rdmas[o] = rdma

        def accum_block(slot, src_idx, first):
            part = jnp.dot(recv_buf[slot],
                           w_ref[pl.ds(src_idx * K_BLK, K_BLK), :],
                           preferred_element_type=jnp.float32)
            if first:
                out_ref[...] = part
            else:
                out_ref[...] += part

        local_cp.wait()
        accum_block(0, my_i, first=True)

        for o in (1, 3, 2):
            rdmas[o].wait_recv()
            src = lax.rem(my_i - o + N_DEV, N_DEV)
            accum_block(o, src, first=False)

        for o in (1, 2, 3):
            rdmas[o].wait_send()

        local_amax = jnp.maximum(jnp.max(out_ref[...]), 0.0)
        amax_src[...] = jnp.full((8, 128), local_amax, jnp.float32)
        a_rdmas = []
        for o in (1, 2, 3):
            t = lax.rem(my_i + o, N_DEV)
            r = pltpu.make_async_remote_copy(
                src_ref=amax_src,
                dst_ref=amax_recv.at[o - 1],
                send_sem=amax_send_sems.at[o - 1],
                recv_sem=amax_recv_sems.at[o - 1],
                device_id=(t,),
                device_id_type=pl.DeviceIdType.MESH)
            r.start()
            a_rdmas.append(r)
        for r in a_rdmas:
            r.wait()
        g_amax = jnp.maximum(local_amax, jnp.max(amax_recv[...]))

        scale = g_amax / 127.0
        inv = 127.0 / g_amax
        y = jnp.maximum(out_ref[...], 0.0)
        q = jnp.clip(jnp.round(y * inv), -127.0, 127.0)
        out_ref[...] = q * scale

    return pl.pallas_call(
        body,
        out_shape=jax.ShapeDtypeStruct((M_BLK, N_OUT), jnp.float32),
        in_specs=[
            pl.BlockSpec(memory_space=pl.ANY),
            pl.BlockSpec(memory_space=pltpu.VMEM),
        ],
        out_specs=pl.BlockSpec(memory_space=pltpu.VMEM),
        scratch_shapes=[
            pltpu.VMEM((N_DEV, M_BLK, K_BLK), jnp.bfloat16),
            pltpu.VMEM((8, 128), jnp.float32),
            pltpu.VMEM((3, 8, 128), jnp.float32),
            pltpu.SemaphoreType.DMA,
            pltpu.SemaphoreType.DMA((3,)),
            pltpu.SemaphoreType.DMA((3,)),
            pltpu.SemaphoreType.DMA((3,)),
            pltpu.SemaphoreType.DMA((3,)),
        ],
        compiler_params=pltpu.CompilerParams(
            collective_id=0, vmem_limit_bytes=100 * 1024 * 1024),
    )(x_bf, w_bf)


# device time: 84151 ns/iter; 2.4516x vs baseline; 1.3683x over previous
import jax
import jax.numpy as jnp
from jax import lax
from jax.experimental import pallas as pl
from jax.experimental.pallas import tpu as pltpu

N_DEV = 4


def kernel(x, w_mat):
    M_BLK = x.shape[0] // N_DEV
    K_BLK = x.shape[1]
    K_ALL = w_mat.shape[0]
    N_OUT = w_mat.shape[1]
    W_CHUNK = min(512, K_BLK)
    NW = K_ALL // W_CHUNK
    N_TILE = min(1024, N_OUT)

    def body(x_hbm, w_hbm, out_ref, x_bf, xstg, w_bf, wstg, recv_buf,
             amax_src, amax_recv, xsems, wsems,
             send_sems, recv_sems, amax_send_sems, amax_recv_sems):
        my_i = lax.axis_index("i")

        barrier = pltpu.get_barrier_semaphore()
        for o in (1, 2, 3):
            peer = lax.rem(my_i + o, N_DEV)
            pl.semaphore_signal(barrier, inc=1, device_id=(peer,),
                                device_id_type=pl.DeviceIdType.MESH)
        pl.semaphore_wait(barrier, N_DEV - 1)

        def xrows(o):
            t = lax.rem(my_i + o, N_DEV)
            return pl.ds(t * M_BLK, M_BLK)

        dests = (2, 1, 3, 0)
        xcps = []
        for idx in range(2):
            cp = pltpu.make_async_copy(
                x_hbm.at[xrows(dests[idx]), :], xstg.at[idx], xsems.at[idx])
            cp.start()
            xcps.append(cp)
        rdmas = {}
        for idx, o in enumerate(dests):
            buf = idx % 2
            xcps[idx].wait()
            x_bf[xrows(o), :] = xstg[buf].astype(jnp.bfloat16)
            if idx + 2 < len(dests):
                cp = pltpu.make_async_copy(
                    x_hbm.at[xrows(dests[idx + 2]), :], xstg.at[buf],
                    xsems.at[buf])
                cp.start()
                xcps.append(cp)
            if o != 0:
                t = lax.rem(my_i + o, N_DEV)
                r = pltpu.make_async_remote_copy(
                    src_ref=x_bf.at[xrows(o), :],
                    dst_ref=recv_buf.at[o - 1],
                    send_sem=send_sems.at[o - 1],
                    recv_sem=recv_sems.at[o - 1],
                    device_id=(t,),
                    device_id_type=pl.DeviceIdType.MESH)
                r.start()
                rdmas[o] = r

        def wrows(c):
            return pl.ds(lax.rem(my_i * K_BLK + c * W_CHUNK, K_ALL), W_CHUNK)

        wcps = []
        for c in range(min(2, NW)):
            cp = pltpu.make_async_copy(
                w_hbm.at[wrows(c), :], wstg.at[c], wsems.at[c])
            cp.start()
            wcps.append(cp)
        for c in range(NW):
            buf = c % 2
            wcps[c].wait()
            w_bf[wrows(c), :] = wstg[buf].astype(jnp.bfloat16)
            if c + 2 < NW:
                cp = pltpu.make_async_copy(
                    w_hbm.at[wrows(c + 2), :], wstg.at[buf], wsems.at[buf])
                cp.start()
                wcps.append(cp)

        def accum_block(x_op, src_idx, first):
            for nt in range(0, N_OUT, N_TILE):
                part = jnp.dot(
                    x_op,
                    w_bf[pl.ds(src_idx * K_BLK, K_BLK), nt:nt + N_TILE],
                    preferred_element_type=jnp.float32)
                if first:
                    out_ref[:, nt:nt + N_TILE] = part
                else:
                    out_ref[:, nt:nt + N_TILE] += part

        accum_block(x_bf[xrows(0), :], my_i, first=True)

        for o in (1, 3, 2):
            rdmas[o].wait_recv()
            src = lax.rem(my_i - o + N_DEV, N_DEV)
            accum_block(recv_buf[o - 1], src, first=False)

        for o in (1, 2, 3):
            rdmas[o].wait_send()

        local_amax = jnp.maximum(jnp.max(out_ref[...]), 0.0)
        amax_src[...] = jnp.full((8, 128), local_amax, jnp.float32)
        a_rdmas = []
        for o in (1, 2, 3):
            t = lax.rem(my_i + o, N_DEV)
            r = pltpu.make_async_remote_copy(
                src_ref=amax_src,
                dst_ref=amax_recv.at[o - 1],
                send_sem=amax_send_sems.at[o - 1],
                recv_sem=amax_recv_sems.at[o - 1],
                device_id=(t,),
                device_id_type=pl.DeviceIdType.MESH)
            r.start()
            a_rdmas.append(r)
        for r in a_rdmas:
            r.wait()
        g_amax = jnp.maximum(local_amax, jnp.max(amax_recv[...]))

        scale = g_amax / 127.0
        inv = 127.0 / g_amax
        y = jnp.maximum(out_ref[...], 0.0)
        q = jnp.clip(jnp.round(y * inv), -127.0, 127.0)
        out_ref[...] = q * scale

    return pl.pallas_call(
        body,
        out_shape=jax.ShapeDtypeStruct((M_BLK, N_OUT), jnp.float32),
        in_specs=[
            pl.BlockSpec(memory_space=pl.ANY),
            pl.BlockSpec(memory_space=pl.ANY),
        ],
        out_specs=pl.BlockSpec(memory_space=pltpu.VMEM),
        scratch_shapes=[
            pltpu.VMEM((N_DEV * M_BLK, K_BLK), jnp.bfloat16),
            pltpu.VMEM((2, M_BLK, K_BLK), jnp.float32),
            pltpu.VMEM((K_ALL, N_OUT), jnp.bfloat16),
            pltpu.VMEM((2, W_CHUNK, N_OUT), jnp.float32),
            pltpu.VMEM((3, M_BLK, K_BLK), jnp.bfloat16),
            pltpu.VMEM((8, 128), jnp.float32),
            pltpu.VMEM((3, 8, 128), jnp.float32),
            pltpu.SemaphoreType.DMA((2,)),
            pltpu.SemaphoreType.DMA((2,)),
            pltpu.SemaphoreType.DMA((3,)),
            pltpu.SemaphoreType.DMA((3,)),
            pltpu.SemaphoreType.DMA((3,)),
            pltpu.SemaphoreType.DMA((3,)),
        ],
        compiler_params=pltpu.CompilerParams(
            collective_id=0, vmem_limit_bytes=100 * 1024 * 1024),
    )(x, w_mat)


# device time: 78033 ns/iter; 2.6438x vs baseline; 1.0784x over previous
import jax
import jax.numpy as jnp
from jax import lax
from jax.experimental import pallas as pl
from jax.experimental.pallas import tpu as pltpu

N_DEV = 4


def kernel(x, w_mat):
    M_BLK = x.shape[0] // N_DEV
    K_BLK = x.shape[1]
    K_ALL = w_mat.shape[0]
    N_OUT = w_mat.shape[1]
    HALF = M_BLK // 2
    W_CHUNK = min(512, K_BLK)
    NW = K_ALL // W_CHUNK
    N_TILE = min(1024, N_OUT)

    XSEQ = ((2, 0), (2, 1), (1, 0), (3, 0), (1, 1), (3, 1), (0, 0), (0, 1))
    RECV_ORDER = ((1, 0), (3, 0), (1, 1), (3, 1), (2, 0), (2, 1))

    def body(x_hbm, w_hbm, out_ref, x_bf, xstg, w_bf, wstg, recv_buf,
             amax_src, amax_recv, xsems, wsems,
             send_sems, recv_sems, amax_send_sems, amax_recv_sems):
        my_i = lax.axis_index("i")

        barrier = pltpu.get_barrier_semaphore()
        for o in (1, 2, 3):
            peer = lax.rem(my_i + o, N_DEV)
            pl.semaphore_signal(barrier, inc=1, device_id=(peer,),
                                device_id_type=pl.DeviceIdType.MESH)
        pl.semaphore_wait(barrier, N_DEV - 1)

        def halfrows(o, h):
            t = lax.rem(my_i + o, N_DEV)
            return pl.ds(t * M_BLK + h * HALF, HALF)

        xcps = []
        for idx in range(2):
            o, h = XSEQ[idx]
            cp = pltpu.make_async_copy(
                x_hbm.at[halfrows(o, h), :], xstg.at[idx], xsems.at[idx])
            cp.start()
            xcps.append(cp)
        rdmas = {}
        for idx, (o, h) in enumerate(XSEQ):
            buf = idx % 2
            xcps[idx].wait()
            x_bf[halfrows(o, h), :] = xstg[buf].astype(jnp.bfloat16)
            if idx + 2 < len(XSEQ):
                o2, h2 = XSEQ[idx + 2]
                cp = pltpu.make_async_copy(
                    x_hbm.at[halfrows(o2, h2), :], xstg.at[buf], xsems.at[buf])
                cp.start()
                xcps.append(cp)
            if o != 0:
                t = lax.rem(my_i + o, N_DEV)
                r = pltpu.make_async_remote_copy(
                    src_ref=x_bf.at[halfrows(o, h), :],
                    dst_ref=recv_buf.at[o - 1, pl.ds(h * HALF, HALF), :],
                    send_sem=send_sems.at[o - 1, h],
                    recv_sem=recv_sems.at[o - 1, h],
                    device_id=(t,),
                    device_id_type=pl.DeviceIdType.MESH)
                r.start()
                rdmas[(o, h)] = r

        def wrows(c):
            return pl.ds(lax.rem(my_i * K_BLK + c * W_CHUNK, K_ALL), W_CHUNK)

        wcps = []
        for c in range(min(2, NW)):
            cp = pltpu.make_async_copy(
                w_hbm.at[wrows(c), :], wstg.at[c], wsems.at[c])
            cp.start()
            wcps.append(cp)
        for c in range(NW):
            buf = c % 2
            wcps[c].wait()
            w_bf[wrows(c), :] = wstg[buf].astype(jnp.bfloat16)
            if c + 2 < NW:
                cp = pltpu.make_async_copy(
                    w_hbm.at[wrows(c + 2), :], wstg.at[buf], wsems.at[buf])
                cp.start()
                wcps.append(cp)

        def accum_half(x_op, src_idx, h, first):
            r0 = h * HALF
            for nt in range(0, N_OUT, N_TILE):
                part = jnp.dot(
                    x_op,
                    w_bf[pl.ds(src_idx * K_BLK, K_BLK), nt:nt + N_TILE],
                    preferred_element_type=jnp.float32)
                if first:
                    out_ref[r0:r0 + HALF, nt:nt + N_TILE] = part
                else:
                    out_ref[r0:r0 + HALF, nt:nt + N_TILE] += part

        for h in (0, 1):
            accum_half(x_bf[halfrows(0, h), :], my_i, h, first=True)

        half_amax = [None, None]
        for o, h in RECV_ORDER:
            rdmas[(o, h)].wait_recv()
            src = lax.rem(my_i - o + N_DEV, N_DEV)
            accum_half(recv_buf[o - 1, h * HALF:(h + 1) * HALF, :],
                       src, h, first=False)
            if o == 2:
                half_amax[h] = jnp.max(out_ref[h * HALF:(h + 1) * HALF, :])

        local_amax = jnp.maximum(jnp.maximum(half_amax[0], half_amax[1]), 0.0)
        amax_src[...] = jnp.full((8, 128), local_amax, jnp.float32)
        a_rdmas = []
        for o in (1, 2, 3):
            t = lax.rem(my_i + o, N_DEV)
            r = pltpu.make_async_remote_copy(
                src_ref=amax_src,
                dst_ref=amax_recv.at[o - 1],
                send_sem=amax_send_sems.at[o - 1],
                recv_sem=amax_recv_sems.at[o - 1],
                device_id=(t,),
                device_id_type=pl.DeviceIdType.MESH)
            r.start()
            a_rdmas.append(r)
        for r in a_rdmas:
            r.wait()
        g_amax = jnp.maximum(local_amax, jnp.max(amax_recv[...]))

        scale = g_amax / 127.0
        inv = 127.0 / g_amax
        y = jnp.maximum(out_ref[...], 0.0)
        q = jnp.clip(jnp.round(y * inv), -127.0, 127.0)
        out_ref[...] = q * scale

        for r in rdmas.values():
            r.wait_send()

    return pl.pallas_call(
        body,
        out_shape=jax.ShapeDtypeStruct((M_BLK, N_OUT), jnp.float32),
        in_specs=[
            pl.BlockSpec(memory_space=pl.ANY),
            pl.BlockSpec(memory_space=pl.ANY),
        ],
        out_specs=pl.BlockSpec(memory_space=pltpu.VMEM),
        scratch_shapes=[
            pltpu.VMEM((N_DEV * M_BLK, K_BLK), jnp.bfloat16),
            pltpu.VMEM((2, HALF, K_BLK), jnp.float32),
            pltpu.VMEM((K_ALL, N_OUT), jnp.bfloat16),
            pltpu.VMEM((2, W_CHUNK, N_OUT), jnp.float32),
            pltpu.VMEM((3, M_BLK, K_BLK), jnp.bfloat16),
            pltpu.VMEM((8, 128), jnp.float32),
            pltpu.VMEM((3, 8, 128), jnp.float32),
            pltpu.SemaphoreType.DMA((2,)),
            pltpu.SemaphoreType.DMA((2,)),
            pltpu.SemaphoreType.DMA((3, 2)),
            pltpu.SemaphoreType.DMA((3, 2)),
            pltpu.SemaphoreType.DMA((3,)),
            pltpu.SemaphoreType.DMA((3,)),
        ],
        compiler_params=pltpu.CompilerParams(
            collective_id=0, vmem_limit_bytes=100 * 1024 * 1024),
    )(x, w_mat)
